# baseline (speedup 1.0000x reference)
import jax
import jax.numpy as jnp
from jax.experimental import pallas as pl
from jax.experimental.pallas import tpu as pltpu

N = 10000
D_IN = 128
D_OUT = 128
M_BLK = 400


def _gcn_kernel(x_ref, w_ref, adj_ref, b_ref, out_ref):
    s = jnp.dot(x_ref[...], w_ref[...], preferred_element_type=jnp.float32)
    out_ref[...] = jnp.dot(adj_ref[...], s,
                           preferred_element_type=jnp.float32) + b_ref[...]


def kernel(x, adj_norm, weight, bias):
    bias2d = bias.reshape(1, D_OUT)
    grid = (pl.cdiv(N, M_BLK),)
    out = pl.pallas_call(
        _gcn_kernel,
        grid=grid,
        in_specs=[
            pl.BlockSpec((N, D_IN), lambda m: (0, 0)),
            pl.BlockSpec((D_IN, D_OUT), lambda m: (0, 0)),
            pl.BlockSpec((M_BLK, N), lambda m: (m, 0)),
            pl.BlockSpec((1, D_OUT), lambda m: (0, 0)),
        ],
        out_specs=pl.BlockSpec((M_BLK, D_OUT), lambda m: (m, 0)),
        out_shape=jax.ShapeDtypeStruct((N, D_OUT), jnp.float32),
        compiler_params=pltpu.CompilerParams(
            dimension_semantics=("parallel",),
        ),
    )(x, weight, adj_norm, bias2d)
    return out


# speedup vs baseline: 1.0022x; 1.0022x over previous
import jax
import jax.numpy as jnp
from jax.experimental import pallas as pl
from jax.experimental.pallas import tpu as pltpu

N = 10000
D_IN = 128
D_OUT = 128
M_BLK = 400


def _gcn_kernel(x_ref, w_ref, adj_ref, b_ref, out_ref, s_ref):
    @pl.when(pl.program_id(0) == 0)
    def _():
        s_ref[...] = jnp.dot(x_ref[...], w_ref[...],
                             preferred_element_type=jnp.float32)

    out_ref[...] = jnp.dot(adj_ref[...], s_ref[...],
                           preferred_element_type=jnp.float32) + b_ref[...]


def kernel(x, adj_norm, weight, bias):
    bias2d = bias.reshape(1, D_OUT)
    grid = (N // M_BLK,)
    out = pl.pallas_call(
        _gcn_kernel,
        grid=grid,
        in_specs=[
            pl.BlockSpec((N, D_IN), lambda m: (0, 0)),
            pl.BlockSpec((D_IN, D_OUT), lambda m: (0, 0)),
            pl.BlockSpec((M_BLK, N), lambda m: (m, 0)),
            pl.BlockSpec((1, D_OUT), lambda m: (0, 0)),
        ],
        out_specs=pl.BlockSpec((M_BLK, D_OUT), lambda m: (m, 0)),
        out_shape=jax.ShapeDtypeStruct((N, D_OUT), jnp.float32),
        scratch_shapes=[pltpu.VMEM((N, D_OUT), jnp.float32)],
        compiler_params=pltpu.CompilerParams(
            dimension_semantics=("arbitrary",),
        ),
    )(x, weight, adj_norm, bias2d)
    return out
